# baseline (speedup 1.0000x reference)
import functools

import jax
import jax.numpy as jnp
from jax import lax
from jax.experimental import pallas as pl
from jax.experimental.pallas import tpu as pltpu
from jax.experimental.pallas import tpu_sc as plsc

K_SEL = 64


def _softmax_kernel(q_ref, k_ref, lm_ref, w_ref):
    q = q_ref[0, 0] * (1.0 / (128.0 ** 0.5))
    s = jax.lax.dot_general(q, k_ref[0, 0], (((1,), (1,)), ((), ())),
                            preferred_element_type=jnp.float32)
    s = s + lm_ref[0, 0]
    m = jnp.max(s, axis=-1, keepdims=True)
    e = jnp.exp(s - m)
    d = jnp.sum(e, axis=-1, keepdims=True)
    w_ref[0, 0] = e * (1.0 / d)


def _topk_mask_kernel(w_ref, wm_ref):
    wb = w_ref[...]
    bsz, n_kv, g, seq = wb.shape
    gsum = jnp.sum(wb, axis=2)
    rows = gsum.reshape(bsz * n_kv, seq)
    gi = jax.lax.bitcast_convert_type(rows, jnp.int32)

    ones = jnp.ones((seq, 128), jnp.float32)
    t = jnp.zeros((gi.shape[0], 1), jnp.int32)
    for bit in range(29, -1, -1):
        cand = t | jnp.int32(1 << bit)
        ind = jnp.where(gi >= cand, jnp.float32(1.0), jnp.float32(0.0))
        cnt = jax.lax.dot_general(ind, ones, (((1,), (0,)), ((), ())),
                                  preferred_element_type=jnp.float32)[:, :1]
        t = jnp.where(cnt >= jnp.float32(K_SEL), cand, t)

    maskf = jnp.where(gi >= t, jnp.float32(1.0), jnp.float32(0.0))
    wm_ref[...] = wb * maskf.reshape(bsz, n_kv, 1, seq)


def _sc_gather_kernel(wm_hbm, v_hbm, vcomp_hbm, wcomp_hbm,
                      wrow_v, idxl_v, idxg_v, vrows_v, wcomp_v, sem):
    rows_per_worker = 8
    n_chunks = 4096 // 16
    wid = lax.axis_index("s") * 2 + lax.axis_index("c")

    for r in range(rows_per_worker):
        row = wid * rows_per_worker + r
        pltpu.sync_copy(wm_hbm.at[row], wrow_v)

        def scan_body(c, pos):
            chunk = wrow_v[0, pl.ds(c * 16, 16)]
            msk = chunk > 0.0
            n = plsc.all_reduce_population_count(msk)

            def have(p):
                ranks = p + plsc.cumsum(jnp.where(msk, jnp.int32(1),
                                                  jnp.int32(0))) - 1
                write = jnp.logical_and(msk, ranks < K_SEL)
                lidx = lax.iota(jnp.int32, 16) + (c * 16)
                plsc.store_scatter(idxl_v, [ranks], lidx, mask=write)
                return p + n

            return lax.cond(n[0] > 0, have, lambda p: p, pos)

        lax.fori_loop(0, n_chunks, scan_body, jnp.zeros((16,), jnp.int32))

        for jc in range(K_SEL // 16):
            idxg_v[pl.ds(jc * 16, 16)] = (idxl_v[pl.ds(jc * 16, 16)]
                                          + (row * 4096))
        pltpu.async_copy(v_hbm.at[idxg_v], vrows_v, sem).wait()

        for g in range(4):
            gv = jnp.full((16,), g, jnp.int32)
            for jc in range(K_SEL // 16):
                idxc = idxl_v[pl.ds(jc * 16, 16)]
                vals = plsc.load_gather(wrow_v, [gv, idxc])
                wcomp_v[g, pl.ds(jc * 16, 16)] = vals

        pltpu.sync_copy(vrows_v, vcomp_hbm.at[row])
        pltpu.sync_copy(wcomp_v, wcomp_hbm.at[row])


def _out_kernel(wc_ref, vc_ref, out_ref):
    for r in range(wc_ref.shape[0]):
        out_ref[r] = jax.lax.dot_general(
            wc_ref[r], vc_ref[r], (((1,), (0,)), ((), ())),
            preferred_element_type=jnp.float32)


def kernel(query, key, value, logmask):
    b, n_heads, _, hs = query.shape
    n_kv = key.shape[1]
    g = n_heads // n_kv
    seq = key.shape[2]
    n_rows = b * n_kv

    q4 = query.reshape(b, n_kv, g, hs)
    lm4 = logmask.reshape(b, n_kv, g, seq)

    w = pl.pallas_call(
        _softmax_kernel,
        grid=(b, n_kv),
        in_specs=[
            pl.BlockSpec((1, 1, g, hs), lambda i, j: (i, j, 0, 0)),
            pl.BlockSpec((1, 1, seq, hs), lambda i, j: (i, j, 0, 0)),
            pl.BlockSpec((1, 1, g, seq), lambda i, j: (i, j, 0, 0)),
        ],
        out_specs=pl.BlockSpec((1, 1, g, seq), lambda i, j: (i, j, 0, 0)),
        out_shape=jax.ShapeDtypeStruct((b, n_kv, g, seq), jnp.float32),
        compiler_params=pltpu.CompilerParams(
            dimension_semantics=("parallel", "parallel")),
    )(q4, key, lm4)

    n_steps = 8
    bb = b // n_steps
    wm = pl.pallas_call(
        _topk_mask_kernel,
        grid=(n_steps,),
        in_specs=[pl.BlockSpec((bb, n_kv, g, seq), lambda i: (i, 0, 0, 0))],
        out_specs=pl.BlockSpec((bb, n_kv, g, seq), lambda i: (i, 0, 0, 0)),
        out_shape=jax.ShapeDtypeStruct((b, n_kv, g, seq), jnp.float32),
        compiler_params=pltpu.CompilerParams(
            dimension_semantics=("parallel",)),
    )(w)

    mesh = plsc.VectorSubcoreMesh(core_axis_name="c", subcore_axis_name="s")
    sc_gather = functools.partial(
        pl.kernel,
        out_type=[
            jax.ShapeDtypeStruct((n_rows, K_SEL, hs), jnp.float32),
            jax.ShapeDtypeStruct((n_rows, g, K_SEL), jnp.float32),
        ],
        mesh=mesh,
        scratch_types=[
            pltpu.VMEM((g, seq), jnp.float32),
            pltpu.VMEM((K_SEL,), jnp.int32),
            pltpu.VMEM((K_SEL,), jnp.int32),
            pltpu.VMEM((K_SEL, hs), jnp.float32),
            pltpu.VMEM((g, K_SEL), jnp.float32),
            pltpu.SemaphoreType.DMA,
        ],
        compiler_params=pltpu.CompilerParams(needs_layout_passes=False),
    )(_sc_gather_kernel)
    vcomp, wcomp = sc_gather(
        wm.reshape(n_rows, g, seq), value.reshape(n_rows * seq, hs))

    rb = 8
    out = pl.pallas_call(
        _out_kernel,
        grid=(n_rows // rb,),
        in_specs=[
            pl.BlockSpec((rb, g, K_SEL), lambda i: (i, 0, 0)),
            pl.BlockSpec((rb, K_SEL, hs), lambda i: (i, 0, 0)),
        ],
        out_specs=pl.BlockSpec((rb, g, hs), lambda i: (i, 0, 0)),
        out_shape=jax.ShapeDtypeStruct((n_rows, g, hs), jnp.float32),
        compiler_params=pltpu.CompilerParams(
            dimension_semantics=("parallel",)),
    )(wcomp, vcomp)

    return out.reshape(b, n_heads, 1, hs), wm.reshape(b, n_heads, 1, seq)


# speedup vs baseline: 1.0230x; 1.0230x over previous
import functools

import jax
import jax.numpy as jnp
from jax import lax
from jax.experimental import pallas as pl
from jax.experimental.pallas import tpu as pltpu
from jax.experimental.pallas import tpu_sc as plsc

K_SEL = 64


def _softmax_kernel(q_ref, k_ref, lm_ref, w_ref):
    q = q_ref[0, 0] * (1.0 / (128.0 ** 0.5))
    s = jax.lax.dot_general(q, k_ref[0, 0], (((1,), (1,)), ((), ())),
                            preferred_element_type=jnp.float32)
    s = s + lm_ref[0, 0]
    m = jnp.max(s, axis=-1, keepdims=True)
    e = jnp.exp(s - m)
    d = jnp.sum(e, axis=-1, keepdims=True)
    w_ref[0, 0] = e * (1.0 / d)


def _topk_mask_kernel(w_ref, wm_ref):
    wb = w_ref[...]
    bsz, n_kv, g, seq = wb.shape
    gsum = jnp.sum(wb, axis=2)
    rows = gsum.reshape(bsz * n_kv, seq)
    gi = jax.lax.bitcast_convert_type(rows, jnp.int32)

    ones = jnp.ones((seq, 128), jnp.float32)
    t = jnp.zeros((gi.shape[0], 1), jnp.int32)
    for bit in range(29, -1, -1):
        cand = t | jnp.int32(1 << bit)
        ind = jnp.where(gi >= cand, jnp.float32(1.0), jnp.float32(0.0))
        cnt = jax.lax.dot_general(ind, ones, (((1,), (0,)), ((), ())),
                                  preferred_element_type=jnp.float32)[:, :1]
        t = jnp.where(cnt >= jnp.float32(K_SEL), cand, t)

    maskf = jnp.where(gi >= t, jnp.float32(1.0), jnp.float32(0.0))
    wm_ref[...] = wb * maskf.reshape(bsz, n_kv, 1, seq)


def _sc_gather_kernel(wm_hbm, v_hbm, vcomp_hbm, wcomp_hbm,
                      wrow_v, idxl_v, idxg_v, vrows_v, wcomp_v, sem):
    rows_per_worker = 8
    n_chunks = 4096 // 16
    wid = lax.axis_index("s") * 2 + lax.axis_index("c")

    for r in range(rows_per_worker):
        row = wid * rows_per_worker + r
        pltpu.sync_copy(wm_hbm.at[row], wrow_v)

        def scan_body(c, pos):
            chunk = wrow_v[0, pl.ds(c * 16, 16)]
            msk = chunk > 0.0
            n = plsc.all_reduce_population_count(msk)
            ranks = pos + plsc.cumsum(jnp.where(msk, jnp.int32(1),
                                                jnp.int32(0))) - 1
            write = jnp.logical_and(msk, ranks < K_SEL)
            lidx = lax.iota(jnp.int32, 16) + (c * 16)
            plsc.store_scatter(idxl_v, [ranks], lidx, mask=write)
            return pos + n

        lax.fori_loop(0, n_chunks, scan_body, jnp.zeros((16,), jnp.int32))

        for jc in range(K_SEL // 16):
            idxg_v[pl.ds(jc * 16, 16)] = (idxl_v[pl.ds(jc * 16, 16)]
                                          + (row * 4096))
        pltpu.async_copy(v_hbm.at[idxg_v], vrows_v, sem).wait()

        for g in range(4):
            gv = jnp.full((16,), g, jnp.int32)
            for jc in range(K_SEL // 16):
                idxc = idxl_v[pl.ds(jc * 16, 16)]
                vals = plsc.load_gather(wrow_v, [gv, idxc])
                wcomp_v[g, pl.ds(jc * 16, 16)] = vals

        pltpu.sync_copy(vrows_v, vcomp_hbm.at[row])
        pltpu.sync_copy(wcomp_v, wcomp_hbm.at[row])


def _out_kernel(wc_ref, vc_ref, out_ref):
    for r in range(wc_ref.shape[0]):
        out_ref[r] = jax.lax.dot_general(
            wc_ref[r], vc_ref[r], (((1,), (0,)), ((), ())),
            preferred_element_type=jnp.float32)


def kernel(query, key, value, logmask):
    b, n_heads, _, hs = query.shape
    n_kv = key.shape[1]
    g = n_heads // n_kv
    seq = key.shape[2]
    n_rows = b * n_kv

    q4 = query.reshape(b, n_kv, g, hs)
    lm4 = logmask.reshape(b, n_kv, g, seq)

    w = pl.pallas_call(
        _softmax_kernel,
        grid=(b, n_kv),
        in_specs=[
            pl.BlockSpec((1, 1, g, hs), lambda i, j: (i, j, 0, 0)),
            pl.BlockSpec((1, 1, seq, hs), lambda i, j: (i, j, 0, 0)),
            pl.BlockSpec((1, 1, g, seq), lambda i, j: (i, j, 0, 0)),
        ],
        out_specs=pl.BlockSpec((1, 1, g, seq), lambda i, j: (i, j, 0, 0)),
        out_shape=jax.ShapeDtypeStruct((b, n_kv, g, seq), jnp.float32),
        compiler_params=pltpu.CompilerParams(
            dimension_semantics=("parallel", "parallel")),
    )(q4, key, lm4)

    n_steps = 8
    bb = b // n_steps
    wm = pl.pallas_call(
        _topk_mask_kernel,
        grid=(n_steps,),
        in_specs=[pl.BlockSpec((bb, n_kv, g, seq), lambda i: (i, 0, 0, 0))],
        out_specs=pl.BlockSpec((bb, n_kv, g, seq), lambda i: (i, 0, 0, 0)),
        out_shape=jax.ShapeDtypeStruct((b, n_kv, g, seq), jnp.float32),
        compiler_params=pltpu.CompilerParams(
            dimension_semantics=("parallel",)),
    )(w)

    mesh = plsc.VectorSubcoreMesh(core_axis_name="c", subcore_axis_name="s")
    sc_gather = functools.partial(
        pl.kernel,
        out_type=[
            jax.ShapeDtypeStruct((n_rows, K_SEL, hs), jnp.float32),
            jax.ShapeDtypeStruct((n_rows, g, K_SEL), jnp.float32),
        ],
        mesh=mesh,
        scratch_types=[
            pltpu.VMEM((g, seq), jnp.float32),
            pltpu.VMEM((K_SEL,), jnp.int32),
            pltpu.VMEM((K_SEL,), jnp.int32),
            pltpu.VMEM((K_SEL, hs), jnp.float32),
            pltpu.VMEM((g, K_SEL), jnp.float32),
            pltpu.SemaphoreType.DMA,
        ],
        compiler_params=pltpu.CompilerParams(needs_layout_passes=False),
    )(_sc_gather_kernel)
    vcomp, wcomp = sc_gather(
        wm.reshape(n_rows, g, seq), value.reshape(n_rows * seq, hs))

    rb = 8
    out = pl.pallas_call(
        _out_kernel,
        grid=(n_rows // rb,),
        in_specs=[
            pl.BlockSpec((rb, g, K_SEL), lambda i: (i, 0, 0)),
            pl.BlockSpec((rb, K_SEL, hs), lambda i: (i, 0, 0)),
        ],
        out_specs=pl.BlockSpec((rb, g, hs), lambda i: (i, 0, 0)),
        out_shape=jax.ShapeDtypeStruct((n_rows, g, hs), jnp.float32),
        compiler_params=pltpu.CompilerParams(
            dimension_semantics=("parallel",)),
    )(wcomp, vcomp)

    return out.reshape(b, n_heads, 1, hs), wm.reshape(b, n_heads, 1, seq)


# speedup vs baseline: 1.1859x; 1.1593x over previous
import functools

import jax
import jax.numpy as jnp
from jax import lax
from jax.experimental import pallas as pl
from jax.experimental.pallas import tpu as pltpu
from jax.experimental.pallas import tpu_sc as plsc

K_SEL = 64


def _softmax_kernel(q_ref, k_ref, lm_ref, w_ref):
    for j in range(k_ref.shape[1]):
        q = q_ref[0, j] * (1.0 / (128.0 ** 0.5))
        s = jax.lax.dot_general(q, k_ref[0, j], (((1,), (1,)), ((), ())),
                                preferred_element_type=jnp.float32)
        s = s + lm_ref[0, j]
        m = jnp.max(s, axis=-1, keepdims=True)
        e = jnp.exp(s - m)
        d = jnp.sum(e, axis=-1, keepdims=True)
        w_ref[0, j] = e * (1.0 / d)


def _topk_mask_kernel(w_ref, wm_ref):
    wb = w_ref[...]
    bsz, n_kv, g, seq = wb.shape
    gsum = jnp.sum(wb, axis=2)
    rows = gsum.reshape(bsz * n_kv, seq)
    gi = jax.lax.bitcast_convert_type(rows, jnp.int32)

    ones = jnp.ones((seq, 128), jnp.float32)
    t = jnp.zeros((gi.shape[0], 1), jnp.int32)
    for bit in range(29, -1, -1):
        cand = t | jnp.int32(1 << bit)
        ind = jnp.where(gi >= cand, jnp.float32(1.0), jnp.float32(0.0))
        cnt = jax.lax.dot_general(ind, ones, (((1,), (0,)), ((), ())),
                                  preferred_element_type=jnp.float32)[:, :1]
        t = jnp.where(cnt >= jnp.float32(K_SEL), cand, t)

    maskf = jnp.where(gi >= t, jnp.float32(1.0), jnp.float32(0.0))
    wm_ref[...] = wb * maskf.reshape(bsz, n_kv, 1, seq)


def _sc_gather_kernel(wm_hbm, v_hbm, vcomp_hbm, wcomp_hbm,
                      wrow_v, idxl_v, idxg_v, vrows_v, wcomp_v, sem):
    rows_per_worker = 8
    n_chunks = 4096 // 16
    wid = lax.axis_index("s") * 2 + lax.axis_index("c")

    for r in range(rows_per_worker):
        row = wid * rows_per_worker + r
        pltpu.sync_copy(wm_hbm.at[row], wrow_v)

        def scan_body(c, pos):
            chunk = wrow_v[0, pl.ds(c * 16, 16)]
            msk = chunk > 0.0
            n = plsc.all_reduce_population_count(msk)
            ranks = pos + plsc.cumsum(jnp.where(msk, jnp.int32(1),
                                                jnp.int32(0))) - 1
            write = jnp.logical_and(msk, ranks < K_SEL)
            lidx = lax.iota(jnp.int32, 16) + (c * 16)
            plsc.store_scatter(idxl_v, [ranks], lidx, mask=write)
            return pos + n

        lax.fori_loop(0, n_chunks, scan_body, jnp.zeros((16,), jnp.int32))

        for jc in range(K_SEL // 16):
            idxg_v[pl.ds(jc * 16, 16)] = (idxl_v[pl.ds(jc * 16, 16)]
                                          + (row * 4096))
        pltpu.async_copy(v_hbm.at[idxg_v], vrows_v, sem).wait()

        for g in range(4):
            gv = jnp.full((16,), g, jnp.int32)
            for jc in range(K_SEL // 16):
                idxc = idxl_v[pl.ds(jc * 16, 16)]
                vals = plsc.load_gather(wrow_v, [gv, idxc])
                wcomp_v[g, pl.ds(jc * 16, 16)] = vals

        pltpu.sync_copy(vrows_v, vcomp_hbm.at[row])
        pltpu.sync_copy(wcomp_v, wcomp_hbm.at[row])


def _out_kernel(wc_ref, vc_ref, out_ref):
    for r in range(wc_ref.shape[0]):
        out_ref[r] = jax.lax.dot_general(
            wc_ref[r], vc_ref[r], (((1,), (0,)), ((), ())),
            preferred_element_type=jnp.float32)


def kernel(query, key, value, logmask):
    b, n_heads, _, hs = query.shape
    n_kv = key.shape[1]
    g = n_heads // n_kv
    seq = key.shape[2]
    n_rows = b * n_kv

    q4 = query.reshape(b, n_kv, g, hs)
    lm4 = logmask.reshape(b, n_kv, g, seq)

    kb = 2
    w = pl.pallas_call(
        _softmax_kernel,
        grid=(b, n_kv // kb),
        in_specs=[
            pl.BlockSpec((1, kb, g, hs), lambda i, j: (i, j, 0, 0)),
            pl.BlockSpec((1, kb, seq, hs), lambda i, j: (i, j, 0, 0)),
            pl.BlockSpec((1, kb, g, seq), lambda i, j: (i, j, 0, 0)),
        ],
        out_specs=pl.BlockSpec((1, kb, g, seq), lambda i, j: (i, j, 0, 0)),
        out_shape=jax.ShapeDtypeStruct((b, n_kv, g, seq), jnp.float32),
        compiler_params=pltpu.CompilerParams(
            dimension_semantics=("parallel", "parallel")),
    )(q4, key, lm4)

    n_steps = 8
    bb = b // n_steps
    wm = pl.pallas_call(
        _topk_mask_kernel,
        grid=(n_steps,),
        in_specs=[pl.BlockSpec((bb, n_kv, g, seq), lambda i: (i, 0, 0, 0))],
        out_specs=pl.BlockSpec((bb, n_kv, g, seq), lambda i: (i, 0, 0, 0)),
        out_shape=jax.ShapeDtypeStruct((b, n_kv, g, seq), jnp.float32),
        compiler_params=pltpu.CompilerParams(
            dimension_semantics=("parallel",)),
    )(w)

    mesh = plsc.VectorSubcoreMesh(core_axis_name="c", subcore_axis_name="s")
    sc_gather = functools.partial(
        pl.kernel,
        out_type=[
            jax.ShapeDtypeStruct((n_rows, K_SEL, hs), jnp.float32),
            jax.ShapeDtypeStruct((n_rows, g, K_SEL), jnp.float32),
        ],
        mesh=mesh,
        scratch_types=[
            pltpu.VMEM((g, seq), jnp.float32),
            pltpu.VMEM((K_SEL,), jnp.int32),
            pltpu.VMEM((K_SEL,), jnp.int32),
            pltpu.VMEM((K_SEL, hs), jnp.float32),
            pltpu.VMEM((g, K_SEL), jnp.float32),
            pltpu.SemaphoreType.DMA,
        ],
        compiler_params=pltpu.CompilerParams(needs_layout_passes=False),
    )(_sc_gather_kernel)
    vcomp, wcomp = sc_gather(
        wm.reshape(n_rows, g, seq), value.reshape(n_rows * seq, hs))

    rb = 8
    out = pl.pallas_call(
        _out_kernel,
        grid=(n_rows // rb,),
        in_specs=[
            pl.BlockSpec((rb, g, K_SEL), lambda i: (i, 0, 0)),
            pl.BlockSpec((rb, K_SEL, hs), lambda i: (i, 0, 0)),
        ],
        out_specs=pl.BlockSpec((rb, g, hs), lambda i: (i, 0, 0)),
        out_shape=jax.ShapeDtypeStruct((n_rows, g, hs), jnp.float32),
        compiler_params=pltpu.CompilerParams(
            dimension_semantics=("parallel",)),
    )(wcomp, vcomp)

    return out.reshape(b, n_heads, 1, hs), wm.reshape(b, n_heads, 1, seq)


# speedup vs baseline: 1.2858x; 1.0842x over previous
import functools

import jax
import jax.numpy as jnp
from jax import lax
from jax.experimental import pallas as pl
from jax.experimental.pallas import tpu as pltpu
from jax.experimental.pallas import tpu_sc as plsc

K_SEL = 64


def _softmax_kernel(q_ref, k_ref, lm_ref, w_ref):
    for j in range(k_ref.shape[1]):
        q = q_ref[0, j] * (1.0 / (128.0 ** 0.5))
        s = jax.lax.dot_general(q, k_ref[0, j], (((1,), (1,)), ((), ())),
                                preferred_element_type=jnp.float32)
        s = s + lm_ref[0, j]
        m = jnp.max(s, axis=-1, keepdims=True)
        e = jnp.exp(s - m)
        d = jnp.sum(e, axis=-1, keepdims=True)
        w_ref[0, j] = e * (1.0 / d)


def _topk_mask_kernel(w_ref, wm_ref):
    wb = w_ref[...]
    bsz, n_kv, g, seq = wb.shape
    gsum = jnp.sum(wb, axis=2)
    rows = gsum.reshape(bsz * n_kv, seq)
    gi = jax.lax.bitcast_convert_type(rows, jnp.int32)

    ones = jnp.ones((seq, 128), jnp.float32)
    t = jnp.zeros((gi.shape[0], 1), jnp.int32)
    for bit in range(29, -1, -1):
        cand = t | jnp.int32(1 << bit)
        ind = jnp.where(gi >= cand, jnp.float32(1.0), jnp.float32(0.0))
        cnt = jax.lax.dot_general(ind, ones, (((1,), (0,)), ((), ())),
                                  preferred_element_type=jnp.float32)[:, :1]
        t = jnp.where(cnt >= jnp.float32(K_SEL), cand, t)

    maskf = jnp.where(gi >= t, jnp.float32(1.0), jnp.float32(0.0))
    wm_ref[...] = wb * maskf.reshape(bsz, n_kv, 1, seq)


def _sc_gather_kernel(wm_hbm, v_hbm, vcomp_hbm, wcomp_hbm,
                      wrow_v, idxl_v, idxg_v, vrows_v, wcomp_v, sem):
    rows_per_worker = 8
    n_chunks = 4096 // 16
    wid = lax.axis_index("s") * 2 + lax.axis_index("c")

    for r in range(rows_per_worker):
        row = wid * rows_per_worker + r
        pltpu.sync_copy(wm_hbm.at[row], wrow_v)

        def scan_body(c, pos):
            chunk = wrow_v[0, pl.ds(c * 16, 16)]
            msk = chunk > 0.0
            n = plsc.all_reduce_population_count(msk)
            ranks = pos + plsc.cumsum(jnp.where(msk, jnp.int32(1),
                                                jnp.int32(0))) - 1
            write = jnp.logical_and(msk, ranks < K_SEL)
            lidx = lax.iota(jnp.int32, 16) + (c * 16)
            plsc.store_scatter(idxl_v, [ranks], lidx, mask=write)
            return pos + n

        lax.fori_loop(0, n_chunks, scan_body, jnp.zeros((16,), jnp.int32))

        for jc in range(K_SEL // 16):
            idxg_v[pl.ds(jc * 16, 16)] = (idxl_v[pl.ds(jc * 16, 16)]
                                          + (row * 4096))
        pltpu.async_copy(v_hbm.at[idxg_v], vrows_v, sem).wait()

        for g in range(4):
            gv = jnp.full((16,), g, jnp.int32)
            for jc in range(K_SEL // 16):
                idxc = idxl_v[pl.ds(jc * 16, 16)]
                vals = plsc.load_gather(wrow_v, [gv, idxc])
                wcomp_v[g, pl.ds(jc * 16, 16)] = vals

        pltpu.sync_copy(vrows_v, vcomp_hbm.at[row])
        pltpu.sync_copy(wcomp_v, wcomp_hbm.at[row])


def _out_kernel(wc_ref, vc_ref, out_ref):
    for r in range(wc_ref.shape[0]):
        out_ref[r] = jax.lax.dot_general(
            wc_ref[r], vc_ref[r], (((1,), (0,)), ((), ())),
            preferred_element_type=jnp.float32)


def kernel(query, key, value, logmask):
    b, n_heads, _, hs = query.shape
    n_kv = key.shape[1]
    g = n_heads // n_kv
    seq = key.shape[2]
    n_rows = b * n_kv

    q4 = query.reshape(b, n_kv, g, hs)
    lm4 = logmask.reshape(b, n_kv, g, seq)

    kb = 4
    w = pl.pallas_call(
        _softmax_kernel,
        grid=(b, n_kv // kb),
        in_specs=[
            pl.BlockSpec((1, kb, g, hs), lambda i, j: (i, j, 0, 0)),
            pl.BlockSpec((1, kb, seq, hs), lambda i, j: (i, j, 0, 0)),
            pl.BlockSpec((1, kb, g, seq), lambda i, j: (i, j, 0, 0)),
        ],
        out_specs=pl.BlockSpec((1, kb, g, seq), lambda i, j: (i, j, 0, 0)),
        out_shape=jax.ShapeDtypeStruct((b, n_kv, g, seq), jnp.float32),
        compiler_params=pltpu.CompilerParams(
            dimension_semantics=("parallel", "parallel")),
    )(q4, key, lm4)

    n_steps = 8
    bb = b // n_steps
    wm = pl.pallas_call(
        _topk_mask_kernel,
        grid=(n_steps,),
        in_specs=[pl.BlockSpec((bb, n_kv, g, seq), lambda i: (i, 0, 0, 0))],
        out_specs=pl.BlockSpec((bb, n_kv, g, seq), lambda i: (i, 0, 0, 0)),
        out_shape=jax.ShapeDtypeStruct((b, n_kv, g, seq), jnp.float32),
        compiler_params=pltpu.CompilerParams(
            dimension_semantics=("parallel",)),
    )(w)

    mesh = plsc.VectorSubcoreMesh(core_axis_name="c", subcore_axis_name="s")
    sc_gather = functools.partial(
        pl.kernel,
        out_type=[
            jax.ShapeDtypeStruct((n_rows, K_SEL, hs), jnp.float32),
            jax.ShapeDtypeStruct((n_rows, g, K_SEL), jnp.float32),
        ],
        mesh=mesh,
        scratch_types=[
            pltpu.VMEM((g, seq), jnp.float32),
            pltpu.VMEM((K_SEL,), jnp.int32),
            pltpu.VMEM((K_SEL,), jnp.int32),
            pltpu.VMEM((K_SEL, hs), jnp.float32),
            pltpu.VMEM((g, K_SEL), jnp.float32),
            pltpu.SemaphoreType.DMA,
        ],
        compiler_params=pltpu.CompilerParams(needs_layout_passes=False),
    )(_sc_gather_kernel)
    vcomp, wcomp = sc_gather(
        wm.reshape(n_rows, g, seq), value.reshape(n_rows * seq, hs))

    rb = 8
    out = pl.pallas_call(
        _out_kernel,
        grid=(n_rows // rb,),
        in_specs=[
            pl.BlockSpec((rb, g, K_SEL), lambda i: (i, 0, 0)),
            pl.BlockSpec((rb, K_SEL, hs), lambda i: (i, 0, 0)),
        ],
        out_specs=pl.BlockSpec((rb, g, hs), lambda i: (i, 0, 0)),
        out_shape=jax.ShapeDtypeStruct((n_rows, g, hs), jnp.float32),
        compiler_params=pltpu.CompilerParams(
            dimension_semantics=("parallel",)),
    )(wcomp, vcomp)

    return out.reshape(b, n_heads, 1, hs), wm.reshape(b, n_heads, 1, seq)


# speedup vs baseline: 1.2867x; 1.0007x over previous
import functools

import jax
import jax.numpy as jnp
from jax import lax
from jax.experimental import pallas as pl
from jax.experimental.pallas import tpu as pltpu
from jax.experimental.pallas import tpu_sc as plsc

K_SEL = 64


def _softmax_kernel(q_ref, k_ref, lm_ref, w_ref):
    for j in range(k_ref.shape[1]):
        q = q_ref[0, j] * (1.0 / (128.0 ** 0.5))
        s = jax.lax.dot_general(q, k_ref[0, j], (((1,), (1,)), ((), ())),
                                preferred_element_type=jnp.float32)
        s = s + lm_ref[0, j]
        m = jnp.max(s, axis=-1, keepdims=True)
        e = jnp.exp(s - m)
        d = jnp.sum(e, axis=-1, keepdims=True)
        w_ref[0, j] = e * (1.0 / d)


def _topk_mask_kernel(w_ref, wm_ref):
    wb = w_ref[...]
    bsz, n_kv, g, seq = wb.shape
    gsum = jnp.sum(wb, axis=2)
    rows = gsum.reshape(bsz * n_kv, seq)
    gi = jax.lax.bitcast_convert_type(rows, jnp.int32)

    ones = jnp.ones((seq, 128), jnp.float32)
    t = jnp.zeros((gi.shape[0], 1), jnp.int32)
    for bit in range(29, -1, -1):
        cand = t | jnp.int32(1 << bit)
        ind = jnp.where(gi >= cand, jnp.float32(1.0), jnp.float32(0.0))
        cnt = jax.lax.dot_general(ind, ones, (((1,), (0,)), ((), ())),
                                  preferred_element_type=jnp.float32)[:, :1]
        t = jnp.where(cnt >= jnp.float32(K_SEL), cand, t)

    maskf = jnp.where(gi >= t, jnp.float32(1.0), jnp.float32(0.0))
    wm_ref[...] = wb * maskf.reshape(bsz, n_kv, 1, seq)


def _sc_gather_kernel(wm_hbm, v_hbm, vcomp_hbm, wcomp_hbm,
                      wrow_v, idxl_v, idxg_v, vrows_v, wcomp_v, sem):
    rows_per_worker = 8
    n_chunks = 4096 // 16
    wid = lax.axis_index("s") * 2 + lax.axis_index("c")

    for r in range(rows_per_worker):
        row = wid * rows_per_worker + r
        pltpu.sync_copy(wm_hbm.at[row], wrow_v)

        def scan_body(c, pos):
            chunk = wrow_v[0, pl.ds(c * 16, 16)]
            msk = chunk > 0.0
            n = plsc.all_reduce_population_count(msk)
            ranks = pos + plsc.cumsum(jnp.where(msk, jnp.int32(1),
                                                jnp.int32(0))) - 1
            write = jnp.logical_and(msk, ranks < K_SEL)
            lidx = lax.iota(jnp.int32, 16) + (c * 16)
            plsc.store_scatter(idxl_v, [ranks], lidx, mask=write)
            return pos + n

        lax.fori_loop(0, n_chunks, scan_body, jnp.zeros((16,), jnp.int32))

        for jc in range(K_SEL // 16):
            idxg_v[pl.ds(jc * 16, 16)] = (idxl_v[pl.ds(jc * 16, 16)]
                                          + (row * 4096))
        pltpu.async_copy(v_hbm.at[idxg_v], vrows_v, sem).wait()

        for g in range(4):
            gv = jnp.full((16,), g, jnp.int32)
            for jc in range(K_SEL // 16):
                idxc = idxl_v[pl.ds(jc * 16, 16)]
                vals = plsc.load_gather(wrow_v, [gv, idxc])
                wcomp_v[g, pl.ds(jc * 16, 16)] = vals

        pltpu.sync_copy(vrows_v, vcomp_hbm.at[row])
        pltpu.sync_copy(wcomp_v, wcomp_hbm.at[row])


def _out_kernel(wc_ref, vc_ref, out_ref):
    for r in range(wc_ref.shape[0]):
        out_ref[r] = jax.lax.dot_general(
            wc_ref[r], vc_ref[r], (((1,), (0,)), ((), ())),
            preferred_element_type=jnp.float32)


def kernel(query, key, value, logmask):
    b, n_heads, _, hs = query.shape
    n_kv = key.shape[1]
    g = n_heads // n_kv
    seq = key.shape[2]
    n_rows = b * n_kv

    q4 = query.reshape(b, n_kv, g, hs)
    lm4 = logmask.reshape(b, n_kv, g, seq)

    kb = 8
    w = pl.pallas_call(
        _softmax_kernel,
        grid=(b, n_kv // kb),
        in_specs=[
            pl.BlockSpec((1, kb, g, hs), lambda i, j: (i, j, 0, 0)),
            pl.BlockSpec((1, kb, seq, hs), lambda i, j: (i, j, 0, 0)),
            pl.BlockSpec((1, kb, g, seq), lambda i, j: (i, j, 0, 0)),
        ],
        out_specs=pl.BlockSpec((1, kb, g, seq), lambda i, j: (i, j, 0, 0)),
        out_shape=jax.ShapeDtypeStruct((b, n_kv, g, seq), jnp.float32),
        compiler_params=pltpu.CompilerParams(
            dimension_semantics=("parallel", "parallel")),
    )(q4, key, lm4)

    n_steps = 8
    bb = b // n_steps
    wm = pl.pallas_call(
        _topk_mask_kernel,
        grid=(n_steps,),
        in_specs=[pl.BlockSpec((bb, n_kv, g, seq), lambda i: (i, 0, 0, 0))],
        out_specs=pl.BlockSpec((bb, n_kv, g, seq), lambda i: (i, 0, 0, 0)),
        out_shape=jax.ShapeDtypeStruct((b, n_kv, g, seq), jnp.float32),
        compiler_params=pltpu.CompilerParams(
            dimension_semantics=("parallel",)),
    )(w)

    mesh = plsc.VectorSubcoreMesh(core_axis_name="c", subcore_axis_name="s")
    sc_gather = functools.partial(
        pl.kernel,
        out_type=[
            jax.ShapeDtypeStruct((n_rows, K_SEL, hs), jnp.float32),
            jax.ShapeDtypeStruct((n_rows, g, K_SEL), jnp.float32),
        ],
        mesh=mesh,
        scratch_types=[
            pltpu.VMEM((g, seq), jnp.float32),
            pltpu.VMEM((K_SEL,), jnp.int32),
            pltpu.VMEM((K_SEL,), jnp.int32),
            pltpu.VMEM((K_SEL, hs), jnp.float32),
            pltpu.VMEM((g, K_SEL), jnp.float32),
            pltpu.SemaphoreType.DMA,
        ],
        compiler_params=pltpu.CompilerParams(needs_layout_passes=False),
    )(_sc_gather_kernel)
    vcomp, wcomp = sc_gather(
        wm.reshape(n_rows, g, seq), value.reshape(n_rows * seq, hs))

    rb = 8
    out = pl.pallas_call(
        _out_kernel,
        grid=(n_rows // rb,),
        in_specs=[
            pl.BlockSpec((rb, g, K_SEL), lambda i: (i, 0, 0)),
            pl.BlockSpec((rb, K_SEL, hs), lambda i: (i, 0, 0)),
        ],
        out_specs=pl.BlockSpec((rb, g, hs), lambda i: (i, 0, 0)),
        out_shape=jax.ShapeDtypeStruct((n_rows, g, hs), jnp.float32),
        compiler_params=pltpu.CompilerParams(
            dimension_semantics=("parallel",)),
    )(wcomp, vcomp)

    return out.reshape(b, n_heads, 1, hs), wm.reshape(b, n_heads, 1, seq)
